# Initial kernel scaffold; baseline (speedup 1.0000x reference)
import functools

import jax
import jax.numpy as jnp
from jax import lax
from jax.experimental import pallas as pl
from jax.experimental.pallas import tpu as pltpu
from jax.experimental.pallas import tpu_sc as plsc

VOCAB = 1000000
EMBED_DIM = 64
HIDDEN_DIM = 128
OUTPUT_DIM = 64
BATCH = 4096
BAG_LEN = 200
TOTAL_TOKENS = BATCH * BAG_LEN

NC = 2
NS = 16
NW = NC * NS

BAGS_PER_CORE = BATCH // NC
BAGS_PER_W = BATCH // NW
TOK_PER_W = BAGS_PER_W * BAG_LEN
TOK_PER_CORE = BAGS_PER_CORE * BAG_LEN

STREAM = 128
CHUNK_T = 1024
N_STREAMS = CHUNK_T // STREAM
N_CHUNKS = TOK_PER_W // CHUNK_T


def _sc_embedding_bag_sums(tokens2d, seg2d, zinit, table):
    mesh = plsc.VectorSubcoreMesh(
        core_axis_name="c", subcore_axis_name="s", num_cores=NC, num_subcores=NS
    )

    @functools.partial(
        pl.kernel,
        mesh=mesh,
        out_type=jax.ShapeDtypeStruct((BATCH, EMBED_DIM), jnp.float32),
        scratch_types=[
            pltpu.VMEM((N_STREAMS, STREAM), jnp.int32),
            pltpu.VMEM((N_STREAMS, STREAM), jnp.int32),
            pltpu.VMEM((CHUNK_T, EMBED_DIM), jnp.float32),
            pltpu.VMEM_SHARED((BAGS_PER_CORE, EMBED_DIM), jnp.float32),
            pltpu.SemaphoreType.DMA,
            pltpu.SemaphoreType.DMA,
        ],
    )
    def body(tok_hbm, seg_hbm, zinit_hbm, table_hbm, out_hbm,
             idx_v, seg_v, rows_v, acc_sh, sem_g, sem_s):
        c = lax.axis_index("c")
        s = lax.axis_index("s")
        acc_base = s * BAGS_PER_W
        pltpu.sync_copy(zinit_hbm, acc_sh.at[pl.ds(acc_base, BAGS_PER_W)])

        tok_row0 = (c * TOK_PER_CORE + s * TOK_PER_W) // STREAM
        seg_row0 = (s * TOK_PER_W) // STREAM

        def chunk(ci, carry):
            pltpu.sync_copy(
                tok_hbm.at[pl.ds(tok_row0 + ci * N_STREAMS, N_STREAMS)], idx_v)
            pltpu.sync_copy(
                seg_hbm.at[pl.ds(seg_row0 + ci * N_STREAMS, N_STREAMS)], seg_v)
            gathers = [
                pltpu.async_copy(
                    table_hbm.at[idx_v.at[j]],
                    rows_v.at[pl.ds(j * STREAM, STREAM)],
                    sem_g,
                )
                for j in range(N_STREAMS)
            ]
            for d in gathers:
                d.wait()
            scatters = [
                pltpu.async_copy(
                    rows_v.at[pl.ds(j * STREAM, STREAM)],
                    acc_sh.at[seg_v.at[j]],
                    sem_s,
                    add=True,
                )
                for j in range(N_STREAMS)
            ]
            for d in scatters:
                d.wait()
            return carry

        lax.fori_loop(0, N_CHUNKS, chunk, 0)

        out_base = c * BAGS_PER_CORE + s * BAGS_PER_W
        pltpu.sync_copy(
            acc_sh.at[pl.ds(acc_base, BAGS_PER_W)],
            out_hbm.at[pl.ds(out_base, BAGS_PER_W)],
        )

    return body(tokens2d, seg2d, zinit, table)


BM = 256


def _mlp_body(x_ref, w1_ref, b1_ref, w2_ref, b2_ref, o_ref):
    x = x_ref[...] * (1.0 / BAG_LEN)
    h = jnp.dot(x, w1_ref[...], preferred_element_type=jnp.float32) + b1_ref[...]
    h = jnp.maximum(h, 0.0)
    p = jnp.dot(h, w2_ref[...], preferred_element_type=jnp.float32) + b2_ref[...]
    norm = jnp.sqrt(jnp.sum(p * p, axis=-1, keepdims=True))
    norm = jnp.maximum(norm, 1e-15)
    maxnorm = 1.0 - 1e-5
    scale = jnp.where(norm > maxnorm, maxnorm / norm, 1.0)
    o_ref[...] = p * scale


def _tc_mlp(sums, W1, b1, W2, b2):
    return pl.pallas_call(
        _mlp_body,
        grid=(BATCH // BM,),
        in_specs=[
            pl.BlockSpec((BM, EMBED_DIM), lambda i: (i, 0)),
            pl.BlockSpec((EMBED_DIM, HIDDEN_DIM), lambda i: (0, 0)),
            pl.BlockSpec((1, HIDDEN_DIM), lambda i: (0, 0)),
            pl.BlockSpec((HIDDEN_DIM, OUTPUT_DIM), lambda i: (0, 0)),
            pl.BlockSpec((1, OUTPUT_DIM), lambda i: (0, 0)),
        ],
        out_specs=pl.BlockSpec((BM, OUTPUT_DIM), lambda i: (i, 0)),
        out_shape=jax.ShapeDtypeStruct((BATCH, OUTPUT_DIM), jnp.float32),
    )(sums, W1, b1, W2, b2)


def kernel(tokens, offsets, table, W1, b1, W2, b2):
    del offsets
    tokens2d = tokens.astype(jnp.int32).reshape(TOTAL_TOKENS // STREAM, STREAM)
    seg = (jnp.arange(TOK_PER_CORE, dtype=jnp.int32) // BAG_LEN).reshape(
        TOK_PER_CORE // STREAM, STREAM)
    zinit = jnp.zeros((BAGS_PER_W, EMBED_DIM), jnp.float32)
    sums = _sc_embedding_bag_sums(tokens2d, seg, zinit, table)
    return _tc_mlp(sums, W1, b1.reshape(1, -1), W2, b2.reshape(1, -1))


# speedup vs baseline: 124.1130x; 124.1130x over previous
import functools

import jax
import jax.numpy as jnp
from jax import lax
from jax.experimental import pallas as pl
from jax.experimental.pallas import tpu as pltpu
from jax.experimental.pallas import tpu_sc as plsc

VOCAB = 1000000
EMBED_DIM = 64
HIDDEN_DIM = 128
OUTPUT_DIM = 64
BATCH = 4096
BAG_LEN = 200
TOTAL_TOKENS = BATCH * BAG_LEN

NC = 2
NS = 16
NW = NC * NS

BAGS_PER_CORE = BATCH // NC
BAGS_PER_W = BATCH // NW
TOK_PER_W = BAGS_PER_W * BAG_LEN
TOK_PER_CORE = BAGS_PER_CORE * BAG_LEN

STREAM = 128
CHUNK_T = 1024
N_STREAMS = CHUNK_T // STREAM
N_CHUNKS = TOK_PER_W // CHUNK_T


def _sc_embedding_bag_sums(tokens2d, seg2d, zinit, table):
    mesh = plsc.VectorSubcoreMesh(
        core_axis_name="c", subcore_axis_name="s", num_cores=NC, num_subcores=NS
    )

    @functools.partial(
        pl.kernel,
        mesh=mesh,
        compiler_params=pltpu.CompilerParams(use_tc_tiling_on_sc=False),
        out_type=jax.ShapeDtypeStruct((BATCH, EMBED_DIM), jnp.float32),
        scratch_types=[
            pltpu.VMEM((N_STREAMS, STREAM), jnp.int32),
            pltpu.VMEM((N_STREAMS, STREAM), jnp.int32),
            pltpu.VMEM((CHUNK_T, EMBED_DIM), jnp.float32),
            pltpu.VMEM_SHARED((BAGS_PER_CORE, EMBED_DIM), jnp.float32),
            pltpu.SemaphoreType.DMA,
            pltpu.SemaphoreType.DMA,
        ],
    )
    def body(tok_hbm, seg_hbm, zinit_hbm, table_hbm, out_hbm,
             idx_v, seg_v, rows_v, acc_sh, sem_g, sem_s):
        c = lax.axis_index("c")
        s = lax.axis_index("s")
        acc_base = s * BAGS_PER_W
        pltpu.sync_copy(zinit_hbm, acc_sh.at[pl.ds(acc_base, BAGS_PER_W)])

        tok_row0 = (c * TOK_PER_CORE + s * TOK_PER_W) // STREAM
        seg_row0 = (s * TOK_PER_W) // STREAM

        def chunk(ci, carry):
            tok_row = pl.multiple_of(tok_row0 + ci * N_STREAMS, N_STREAMS)
            seg_row = pl.multiple_of(seg_row0 + ci * N_STREAMS, N_STREAMS)
            pltpu.sync_copy(tok_hbm.at[pl.ds(tok_row, N_STREAMS)], idx_v)
            pltpu.sync_copy(seg_hbm.at[pl.ds(seg_row, N_STREAMS)], seg_v)
            gathers = [
                pltpu.async_copy(
                    table_hbm.at[idx_v.at[j]],
                    rows_v.at[pl.ds(j * STREAM, STREAM)],
                    sem_g,
                )
                for j in range(N_STREAMS)
            ]
            for d in gathers:
                d.wait()
            scatters = [
                pltpu.async_copy(
                    rows_v.at[pl.ds(j * STREAM, STREAM)],
                    acc_sh.at[seg_v.at[j]],
                    sem_s,
                    add=True,
                )
                for j in range(N_STREAMS)
            ]
            for d in scatters:
                d.wait()
            return carry

        lax.fori_loop(0, N_CHUNKS, chunk, 0)

        out_base = pl.multiple_of(c * BAGS_PER_CORE + s * BAGS_PER_W, BAGS_PER_W)
        pltpu.sync_copy(
            acc_sh.at[pl.ds(acc_base, BAGS_PER_W)],
            out_hbm.at[pl.ds(out_base, BAGS_PER_W)],
        )

    return body(tokens2d, seg2d, zinit, table)


BM = 256


def _mlp_body(x_ref, w1_ref, b1_ref, w2_ref, b2_ref, o_ref):
    x = x_ref[...] * (1.0 / BAG_LEN)
    h = jnp.dot(x, w1_ref[...], preferred_element_type=jnp.float32) + b1_ref[...]
    h = jnp.maximum(h, 0.0)
    p = jnp.dot(h, w2_ref[...], preferred_element_type=jnp.float32) + b2_ref[...]
    norm = jnp.sqrt(jnp.sum(p * p, axis=-1, keepdims=True))
    norm = jnp.maximum(norm, 1e-15)
    maxnorm = 1.0 - 1e-5
    scale = jnp.where(norm > maxnorm, maxnorm / norm, 1.0)
    o_ref[...] = p * scale


def _tc_mlp(sums, W1, b1, W2, b2):
    return pl.pallas_call(
        _mlp_body,
        grid=(BATCH // BM,),
        in_specs=[
            pl.BlockSpec((BM, EMBED_DIM), lambda i: (i, 0)),
            pl.BlockSpec((EMBED_DIM, HIDDEN_DIM), lambda i: (0, 0)),
            pl.BlockSpec((1, HIDDEN_DIM), lambda i: (0, 0)),
            pl.BlockSpec((HIDDEN_DIM, OUTPUT_DIM), lambda i: (0, 0)),
            pl.BlockSpec((1, OUTPUT_DIM), lambda i: (0, 0)),
        ],
        out_specs=pl.BlockSpec((BM, OUTPUT_DIM), lambda i: (i, 0)),
        out_shape=jax.ShapeDtypeStruct((BATCH, OUTPUT_DIM), jnp.float32),
    )(sums, W1, b1, W2, b2)


def kernel(tokens, offsets, table, W1, b1, W2, b2):
    del offsets
    tokens2d = tokens.astype(jnp.int32).reshape(TOTAL_TOKENS // STREAM, STREAM)
    seg = (jnp.arange(TOK_PER_CORE, dtype=jnp.int32) // BAG_LEN).reshape(
        TOK_PER_CORE // STREAM, STREAM)
    zinit = jnp.zeros((BAGS_PER_W, EMBED_DIM), jnp.float32)
    sums = _sc_embedding_bag_sums(tokens2d, seg, zinit, table)
    return _tc_mlp(sums, W1, b1.reshape(1, -1), W2, b2.reshape(1, -1))


# speedup vs baseline: 127.2392x; 1.0252x over previous
import functools

import jax
import jax.numpy as jnp
from jax import lax
from jax.experimental import pallas as pl
from jax.experimental.pallas import tpu as pltpu
from jax.experimental.pallas import tpu_sc as plsc

VOCAB = 1000000
EMBED_DIM = 64
HIDDEN_DIM = 128
OUTPUT_DIM = 64
BATCH = 4096
BAG_LEN = 200
TOTAL_TOKENS = BATCH * BAG_LEN

NC = 2
NS = 16
NW = NC * NS

BAGS_PER_CORE = BATCH // NC
BAGS_PER_W = BATCH // NW
TOK_PER_W = BAGS_PER_W * BAG_LEN
TOK_PER_CORE = BAGS_PER_CORE * BAG_LEN

STREAM = 128
CHUNK_T = 512
N_STREAMS = CHUNK_T // STREAM
N_CHUNKS = TOK_PER_W // CHUNK_T
IDX_ROWS = TOK_PER_W // STREAM


def _sc_embedding_bag_sums(tokens2d, seg2d, zinit, table):
    mesh = plsc.VectorSubcoreMesh(
        core_axis_name="c", subcore_axis_name="s", num_cores=NC, num_subcores=NS
    )

    @functools.partial(
        pl.kernel,
        mesh=mesh,
        compiler_params=pltpu.CompilerParams(use_tc_tiling_on_sc=False),
        out_type=jax.ShapeDtypeStruct((BATCH, EMBED_DIM), jnp.float32),
        scratch_types=[
            pltpu.VMEM((IDX_ROWS, STREAM), jnp.int32),
            pltpu.VMEM((IDX_ROWS, STREAM), jnp.int32),
            pltpu.VMEM((2, CHUNK_T, EMBED_DIM), jnp.float32),
            pltpu.VMEM_SHARED((BAGS_PER_CORE, EMBED_DIM), jnp.float32),
            pltpu.SemaphoreType.DMA,
            pltpu.SemaphoreType.DMA,
        ],
    )
    def body(tok_hbm, seg_hbm, zinit_hbm, table_hbm, out_hbm,
             idx_all, seg_all, rows_v, acc_sh, sem_g, sem_s):
        c = lax.axis_index("c")
        s = lax.axis_index("s")
        acc_base = s * BAGS_PER_W
        pltpu.sync_copy(zinit_hbm, acc_sh.at[pl.ds(acc_base, BAGS_PER_W)])
        tok_row0 = pl.multiple_of(
            (c * TOK_PER_CORE + s * TOK_PER_W) // STREAM, N_STREAMS)
        seg_row0 = pl.multiple_of((s * TOK_PER_W) // STREAM, N_STREAMS)
        pltpu.sync_copy(tok_hbm.at[pl.ds(tok_row0, IDX_ROWS)], idx_all)
        pltpu.sync_copy(seg_hbm.at[pl.ds(seg_row0, IDX_ROWS)], seg_all)

        def fire_gathers(ci, b):
            for j in range(N_STREAMS):
                pltpu.async_copy(
                    table_hbm.at[idx_all.at[ci * N_STREAMS + j]],
                    rows_v.at[b, pl.ds(j * STREAM, STREAM)],
                    sem_g,
                )

        def wait_gathers(b):
            for j in range(N_STREAMS):
                pltpu.make_async_copy(
                    table_hbm.at[idx_all.at[0]],
                    rows_v.at[b, pl.ds(j * STREAM, STREAM)],
                    sem_g,
                ).wait()

        def fire_scatters(ci, b):
            for j in range(N_STREAMS):
                pltpu.async_copy(
                    rows_v.at[b, pl.ds(j * STREAM, STREAM)],
                    acc_sh.at[seg_all.at[ci * N_STREAMS + j]],
                    sem_s,
                    add=True,
                )

        def wait_scatters(b):
            for j in range(N_STREAMS):
                pltpu.make_async_copy(
                    rows_v.at[b, pl.ds(j * STREAM, STREAM)],
                    acc_sh.at[seg_all.at[0]],
                    sem_s,
                ).wait()

        fire_gathers(0, 0)
        def chunk(ci, carry):
            b = lax.rem(ci, 2)
            wait_gathers(b)
            fire_scatters(ci, b)
            wait_scatters(b)
            fire_gathers(ci + 1, 1 - b)
            return carry
        lax.fori_loop(0, N_CHUNKS - 1, chunk, 0)
        b_last = (N_CHUNKS - 1) % 2
        wait_gathers(b_last)
        fire_scatters(N_CHUNKS - 1, b_last)
        wait_scatters(b_last)

        out_base = pl.multiple_of(c * BAGS_PER_CORE + s * BAGS_PER_W, BAGS_PER_W)
        pltpu.sync_copy(
            acc_sh.at[pl.ds(acc_base, BAGS_PER_W)],
            out_hbm.at[pl.ds(out_base, BAGS_PER_W)],
        )

    return body(tokens2d, seg2d, zinit, table)


BM = 256


def _mlp_body(x_ref, w1_ref, b1_ref, w2_ref, b2_ref, o_ref):
    x = x_ref[...] * (1.0 / BAG_LEN)
    h = jnp.dot(x, w1_ref[...], preferred_element_type=jnp.float32) + b1_ref[...]
    h = jnp.maximum(h, 0.0)
    p = jnp.dot(h, w2_ref[...], preferred_element_type=jnp.float32) + b2_ref[...]
    norm = jnp.sqrt(jnp.sum(p * p, axis=-1, keepdims=True))
    norm = jnp.maximum(norm, 1e-15)
    maxnorm = 1.0 - 1e-5
    scale = jnp.where(norm > maxnorm, maxnorm / norm, 1.0)
    o_ref[...] = p * scale


def _tc_mlp(sums, W1, b1, W2, b2):
    return pl.pallas_call(
        _mlp_body,
        grid=(BATCH // BM,),
        in_specs=[
            pl.BlockSpec((BM, EMBED_DIM), lambda i: (i, 0)),
            pl.BlockSpec((EMBED_DIM, HIDDEN_DIM), lambda i: (0, 0)),
            pl.BlockSpec((1, HIDDEN_DIM), lambda i: (0, 0)),
            pl.BlockSpec((HIDDEN_DIM, OUTPUT_DIM), lambda i: (0, 0)),
            pl.BlockSpec((1, OUTPUT_DIM), lambda i: (0, 0)),
        ],
        out_specs=pl.BlockSpec((BM, OUTPUT_DIM), lambda i: (i, 0)),
        out_shape=jax.ShapeDtypeStruct((BATCH, OUTPUT_DIM), jnp.float32),
    )(sums, W1, b1, W2, b2)


def kernel(tokens, offsets, table, W1, b1, W2, b2):
    del offsets
    tokens2d = tokens.astype(jnp.int32).reshape(TOTAL_TOKENS // STREAM, STREAM)
    seg = (jnp.arange(TOK_PER_CORE, dtype=jnp.int32) // BAG_LEN).reshape(
        TOK_PER_CORE // STREAM, STREAM)
    zinit = jnp.zeros((BAGS_PER_W, EMBED_DIM), jnp.float32)
    sums = _sc_embedding_bag_sums(tokens2d, seg, zinit, table)
    return _tc_mlp(sums, W1, b1.reshape(1, -1), W2, b2.reshape(1, -1))


# speedup vs baseline: 183.0514x; 1.4386x over previous
import functools

import jax
import jax.numpy as jnp
from jax import lax
from jax.experimental import pallas as pl
from jax.experimental.pallas import tpu as pltpu
from jax.experimental.pallas import tpu_sc as plsc

VOCAB = 1000000
EMBED_DIM = 64
HIDDEN_DIM = 128
OUTPUT_DIM = 64
BATCH = 4096
BAG_LEN = 200
TOTAL_TOKENS = BATCH * BAG_LEN

NC = 2
NS = 16
NW = NC * NS

BAGS_PER_CORE = BATCH // NC
BAGS_PER_W = BATCH // NW
TOK_PER_W = BAGS_PER_W * BAG_LEN
TOK_PER_CORE = BAGS_PER_CORE * BAG_LEN

STREAM = 128
CHUNK_T = 512
N_STREAMS = CHUNK_T // STREAM
N_CHUNKS = TOK_PER_W // CHUNK_T
IDX_ROWS = TOK_PER_W // STREAM
SEG_PERIOD = 3200 // STREAM


def _sc_embedding_bag_sums(tokens2d, seg2d, zinit, table):
    mesh = plsc.VectorSubcoreMesh(
        core_axis_name="c", subcore_axis_name="s", num_cores=NC, num_subcores=NS
    )

    @functools.partial(
        pl.kernel,
        mesh=mesh,
        compiler_params=pltpu.CompilerParams(use_tc_tiling_on_sc=False),
        out_type=jax.ShapeDtypeStruct((BATCH, EMBED_DIM), jnp.float32),
        scratch_types=[
            pltpu.VMEM((IDX_ROWS, STREAM), jnp.int32),
            pltpu.VMEM((SEG_PERIOD, STREAM), jnp.int32),
            pltpu.VMEM((2, N_STREAMS, STREAM), jnp.int32),
            pltpu.VMEM((2, CHUNK_T, EMBED_DIM), jnp.float32),
            pltpu.VMEM_SHARED((2, BAGS_PER_CORE, EMBED_DIM), jnp.float32),
            pltpu.SemaphoreType.DMA,
            pltpu.SemaphoreType.DMA,
        ],
    )
    def body(tok_hbm, seg_hbm, zinit_hbm, table_hbm, out_hbm,
             idx_all, segb_v, seg_use, rows_v, acc_sh, sem_g, sem_s):
        c = lax.axis_index("c")
        s = lax.axis_index("s")
        acc_base = s * BAGS_PER_W
        pltpu.sync_copy(zinit_hbm, acc_sh.at[0, pl.ds(acc_base, BAGS_PER_W)])
        pltpu.sync_copy(zinit_hbm, acc_sh.at[1, pl.ds(acc_base, BAGS_PER_W)])
        tok_row0 = pl.multiple_of(
            (c * TOK_PER_CORE + s * TOK_PER_W) // STREAM, N_STREAMS)
        pltpu.sync_copy(tok_hbm.at[pl.ds(tok_row0, IDX_ROWS)], idx_all)
        pltpu.sync_copy(seg_hbm, segb_v)

        def compute_seg(ci, b):
            for j in range(N_STREAMS):
                r = ci * N_STREAMS + j
                q = r // SEG_PERIOD
                rr = r % SEG_PERIOD
                addv = s * BAGS_PER_W + 16 * q
                for k in range(STREAM // 16):
                    seg_use[b, j, pl.ds(k * 16, 16)] = (
                        segb_v[rr, pl.ds(k * 16, 16)] + addv)

        def fire_gathers(ci, b):
            for j in range(N_STREAMS):
                pltpu.async_copy(
                    table_hbm.at[idx_all.at[ci * N_STREAMS + j]],
                    rows_v.at[b, pl.ds(j * STREAM, STREAM)],
                    sem_g,
                )

        def wait_gathers(b):
            for j in range(N_STREAMS):
                pltpu.make_async_copy(
                    table_hbm.at[idx_all.at[0]],
                    rows_v.at[b, pl.ds(j * STREAM, STREAM)],
                    sem_g,
                ).wait()

        def fire_scatters(ci, b):
            for j in range(N_STREAMS):
                pltpu.async_copy(
                    rows_v.at[b, pl.ds(j * STREAM, STREAM)],
                    acc_sh.at[j % 2].at[seg_use.at[b, j]],
                    sem_s,
                    add=True,
                ).wait()

        fire_gathers(0, 0)
        def chunk(ci, carry):
            b = lax.rem(ci, 2)
            wait_gathers(b)
            compute_seg(ci, b)
            fire_scatters(ci, b)
            fire_gathers(ci + 1, 1 - b)
            return carry
        lax.fori_loop(0, N_CHUNKS - 1, chunk, 0)
        b_last = (N_CHUNKS - 1) % 2
        wait_gathers(b_last)
        compute_seg(N_CHUNKS - 1, b_last)
        fire_scatters(N_CHUNKS - 1, b_last)

        pltpu.sync_copy(acc_sh.at[0, pl.ds(acc_base, BAGS_PER_W)],
                        rows_v.at[0, pl.ds(0, BAGS_PER_W)])
        pltpu.sync_copy(acc_sh.at[1, pl.ds(acc_base, BAGS_PER_W)],
                        rows_v.at[1, pl.ds(0, BAGS_PER_W)])

        def combine(i, carry):
            for k in range(EMBED_DIM // 16):
                rows_v[0, i, pl.ds(k * 16, 16)] = (
                    rows_v[0, i, pl.ds(k * 16, 16)]
                    + rows_v[1, i, pl.ds(k * 16, 16)])
            return carry
        lax.fori_loop(0, BAGS_PER_W, combine, 0)

        out_base = pl.multiple_of(c * BAGS_PER_CORE + s * BAGS_PER_W, BAGS_PER_W)
        pltpu.sync_copy(
            rows_v.at[0, pl.ds(0, BAGS_PER_W)],
            out_hbm.at[pl.ds(out_base, BAGS_PER_W)],
        )

    return body(tokens2d, seg2d, zinit, table)


REPACK_NB = 4096
REPACK_G = -(-VOCAB // REPACK_NB)
VROWS = REPACK_G * REPACK_NB


def _repack_body(in_ref, out_ref):
    xT = in_ref[...].T
    out_ref[...] = jnp.concatenate(
        [xT[: REPACK_NB // 2], xT[REPACK_NB // 2 :]], axis=1)


def _tc_repack(tableT):
    return pl.pallas_call(
        _repack_body,
        grid=(REPACK_G,),
        in_specs=[pl.BlockSpec((EMBED_DIM, REPACK_NB), lambda i: (0, i))],
        out_specs=pl.BlockSpec((REPACK_NB // 2, 2 * EMBED_DIM), lambda i: (i, 0)),
        out_shape=jax.ShapeDtypeStruct(
            (REPACK_G * REPACK_NB // 2, 2 * EMBED_DIM), jnp.float32),
    )(tableT)


BM = 256


def _mlp_body(x_ref, w1_ref, b1_ref, w2_ref, b2_ref, o_ref):
    x = x_ref[...] * (1.0 / BAG_LEN)
    h = jnp.dot(x, w1_ref[...], preferred_element_type=jnp.float32) + b1_ref[...]
    h = jnp.maximum(h, 0.0)
    p = jnp.dot(h, w2_ref[...], preferred_element_type=jnp.float32) + b2_ref[...]
    norm = jnp.sqrt(jnp.sum(p * p, axis=-1, keepdims=True))
    norm = jnp.maximum(norm, 1e-15)
    maxnorm = 1.0 - 1e-5
    scale = jnp.where(norm > maxnorm, maxnorm / norm, 1.0)
    o_ref[...] = p * scale


def _tc_mlp(sums, W1, b1, W2, b2):
    return pl.pallas_call(
        _mlp_body,
        grid=(BATCH // BM,),
        in_specs=[
            pl.BlockSpec((BM, EMBED_DIM), lambda i: (i, 0)),
            pl.BlockSpec((EMBED_DIM, HIDDEN_DIM), lambda i: (0, 0)),
            pl.BlockSpec((1, HIDDEN_DIM), lambda i: (0, 0)),
            pl.BlockSpec((HIDDEN_DIM, OUTPUT_DIM), lambda i: (0, 0)),
            pl.BlockSpec((1, OUTPUT_DIM), lambda i: (0, 0)),
        ],
        out_specs=pl.BlockSpec((BM, OUTPUT_DIM), lambda i: (i, 0)),
        out_shape=jax.ShapeDtypeStruct((BATCH, OUTPUT_DIM), jnp.float32),
    )(sums, W1, b1, W2, b2)


def kernel(tokens, offsets, table, W1, b1, W2, b2):
    del offsets
    t32 = tokens.astype(jnp.int32)
    g = t32 // REPACK_NB
    r = t32 % REPACK_NB
    nidx = g * REPACK_NB + 2 * (r & (REPACK_NB // 2 - 1)) + (r >> 11)
    tokens2d = nidx.reshape(TOTAL_TOKENS // STREAM, STREAM)
    seg = (jnp.arange(SEG_PERIOD * STREAM, dtype=jnp.int32) // BAG_LEN).reshape(
        SEG_PERIOD, STREAM)
    zinit = jnp.zeros((BAGS_PER_W, EMBED_DIM), jnp.float32)
    table_lin = _tc_repack(table.T).reshape(VROWS, EMBED_DIM)
    sums = _sc_embedding_bag_sums(tokens2d, seg, zinit, table_lin)
    return _tc_mlp(sums, W1, b1.reshape(1, -1), W2, b2.reshape(1, -1))
